# baseline (speedup 1.0000x reference)
import functools

import numpy as np
import jax
import jax.numpy as jnp
from jax.experimental import pallas as pl
from jax.experimental.pallas import tpu as pltpu

_EPS = 1e-5
_B = 64
_CH = 2
_H = _W = 32


def _tap_masks(H, W, B):
    hw = H * W
    r = np.arange(hw) // W
    c = np.arange(hw) % W
    ms = []
    for dy in (-1, 0, 1):
        for dx in (-1, 0, 1):
            m = ((r + dy >= 0) & (r + dy <= H - 1)
                 & (c + dx >= 0) & (c + dx <= W - 1)).astype(np.float32)
            ms.append(np.tile(m, B))
    return np.stack(ms)


def _image_ones(hw, B):
    o = np.zeros((B * hw, B), np.float32)
    for b in range(B):
        o[b * hw:(b + 1) * hw, b] = 1.0
    return o


def _pool_sel(H, W):
    hw, wo = H * W, W // 2
    hwo = (H // 2) * wo
    sel = np.zeros((hw, hwo), np.float32)
    for q in range(hwo):
        p = (2 * (q // wo)) * W + 2 * (q % wo)
        sel[p, q] = 1.0
    return sel


_MA = _tap_masks(_H, _W, _B)
_MB = _tap_masks(_H // 2, _W // 2, _B)
_MC = _tap_masks(_H // 4, _W // 4, _B)
_OA = _image_ones(_H * _W, _B)
_OB = _image_ones((_H // 2) * (_W // 2), _B)
_OC = _image_ones((_H // 4) * (_W // 4), _B)
_S1 = _pool_sel(_H, _W)
_S2 = _pool_sel(_H // 2, _W // 2)


def _roll(x, s):
    L = x.shape[-1]
    s = s % L
    if s == 0:
        return x
    return pltpu.roll(x, shift=L - s, axis=x.ndim - 1)


def _net_kernel(x_ref, w1, w2, w3, w4, w5, w6, w7, w8, w9, w10,
                gb, pA, pB, pC, pD, pE, mA, mB, mC, oA, oB, oC, s1, s2,
                o_ref, *, B, CH):
    f32, bf16 = jnp.float32, jnp.bfloat16
    Bc = B // CH

    def expand(v, hw):
        parts = [jnp.broadcast_to(v[:, b:b + 1], (v.shape[0], hw))
                 for b in range(Bc)]
        return jnp.concatenate(parts, axis=1)

    def taps(x, msk, w_stage):
        pieces = []
        for dy in (-1, 0, 1):
            for dx in (-1, 0, 1):
                t = 3 * (dy + 1) + (dx + 1)
                v = _roll(x, dy * w_stage + dx)
                if dy != 0 or dx != 0:
                    v = v * msk[t:t + 1, :]
                pieces.append(v)
        return jnp.concatenate(pieces, axis=0)

    def conv_gn(x, wc_ref, p_ref, gn_idx, msk, onesb, hw, w_stage):
        cout = wc_ref.shape[0]
        y = jnp.dot(wc_ref[...], taps(x, msk, w_stage),
                    preferred_element_type=f32)
        yb = jnp.maximum(y.astype(bf16), 0)
        ys = jnp.concatenate([yb, yb * yb], axis=0)
        S2 = jnp.dot(ys, onesb, preferred_element_type=f32)
        stats = jnp.dot(p_ref[...],
                        jnp.concatenate([S2[:cout], S2[cout:]], axis=1),
                        preferred_element_type=f32)
        mean, e2 = stats[:, :Bc], stats[:, Bc:]
        var = jnp.maximum(e2 - mean * mean, 0.0)
        inv = jax.lax.rsqrt(var + _EPS)
        gamma = gb[0:cout, 2 * gn_idx:2 * gn_idx + 1]
        beta = gb[0:cout, 2 * gn_idx + 1:2 * gn_idx + 2]
        scale = inv * gamma
        offset = beta - mean * scale
        return yb * expand(scale.astype(bf16), hw) \
            + expand(offset.astype(bf16), hw)

    def maxpool(x, sel_ref, hw_in, w_stage):
        m1 = jnp.maximum(x, _roll(x, 1))
        m2 = jnp.maximum(m1, _roll(m1, w_stage))
        outs = [jnp.dot(m2[:, b * hw_in:(b + 1) * hw_in], sel_ref[...],
                        preferred_element_type=f32) for b in range(Bc)]
        return jnp.concatenate(outs, axis=1).astype(bf16)

    hwA, hwB, hwC = _H * _W, (_H // 2) * (_W // 2), (_H // 4) * (_W // 4)

    mAc, mBc, mCc = mA[:, :Bc * hwA], mB[:, :Bc * hwB], mC[:, :Bc * hwC]
    oAc, oBc, oCc = oA[:Bc * hwA, :Bc], oB[:Bc * hwB, :Bc], oC[:Bc * hwC, :Bc]

    def layer(fn, xs):
        return [fn(x) for x in xs]

    xin = x_ref[0]
    xs = [xin[:, c * Bc * hwA:(c + 1) * Bc * hwA] for c in range(CH)]
    xs = layer(lambda x: conv_gn(x, w1, pA, 0, mAc, oAc, hwA, _W), xs)
    xs = layer(lambda x: conv_gn(x, w2, pA, 1, mAc, oAc, hwA, _W), xs)
    xs = layer(lambda x: jnp.dot(w3[...], x,
                                 preferred_element_type=f32).astype(bf16), xs)
    xs = layer(lambda x: maxpool(x, s1, hwA, _W), xs)
    xs = layer(lambda x: conv_gn(x, w4, pB, 2, mBc, oBc, hwB, _W // 2), xs)
    xs = layer(lambda x: conv_gn(x, w5, pB, 3, mBc, oBc, hwB, _W // 2), xs)
    xs = layer(lambda x: jnp.dot(w6[...], x,
                                 preferred_element_type=f32).astype(bf16), xs)
    xs = layer(lambda x: maxpool(x, s2, hwB, _W // 2), xs)
    xs = layer(lambda x: conv_gn(x, w7, pC, 4, mCc, oCc, hwC, _W // 4), xs)
    xs = layer(lambda x: conv_gn(x, w8, pD, 5, mCc, oCc, hwC, _W // 4), xs)
    xs = layer(lambda x: conv_gn(x, w9, pE, 6, mCc, oCc, hwC, _W // 4), xs)

    def head(zf):
        g = jnp.dot(zf, oCc,
                    preferred_element_type=f32) * (1.0 / hwC)
        z = jnp.dot(w10[...], g, preferred_element_type=f32)
        m = jnp.max(z, axis=0, keepdims=True)
        lse = jnp.log(jnp.sum(jnp.exp(z - m), axis=0, keepdims=True)) + m
        return z - lse

    zs = layer(head, xs)
    o_ref[0] = zs[0] if CH == 1 else jnp.concatenate(zs, axis=1)


def _const_spec(a):
    zeros = (0,) * a.ndim
    return pl.BlockSpec(a.shape, lambda n, _z=zeros: _z)


def kernel(x, op00, op01, op02, op03, op04, op05, op06, op07, op08, op09,
           op10, op11, op12, op13, op14, op15, op16, op17, op18, op19):
    B, H, W = _B, _H, _W
    N = x.shape[0]
    G = N // B
    bf16 = jnp.bfloat16

    def wc(op):
        c9, cout, cin = op.shape
        return jnp.transpose(op, (1, 0, 2)).reshape(cout, c9 * cin).astype(bf16)

    ops = [
        wc(op00), wc(op01), op02.astype(bf16),
        wc(op03), wc(op04), op05.astype(bf16),
        wc(op06), wc(op07), wc(op08),
        op09,
        op10,
        op11, op12, op13, op14, op15,
        jnp.asarray(_MA, bf16), jnp.asarray(_MB, bf16), jnp.asarray(_MC, bf16),
        jnp.asarray(_OA, bf16), jnp.asarray(_OB, bf16), jnp.asarray(_OC, bf16),
        jnp.asarray(_S1, bf16), jnp.asarray(_S2, bf16),
    ]

    xp = jnp.pad(x.astype(jnp.float32), ((0, 0), (0, 8 - 3), (0, 0), (0, 0)))
    xp = xp.reshape(G, B, 8, H * W)
    xp = jnp.transpose(xp, (0, 2, 1, 3)).reshape(G, 8, B * H * W).astype(bf16)

    in_specs = [pl.BlockSpec((1, 8, B * H * W), lambda n: (n, 0, 0))]
    in_specs += [_const_spec(a) for a in ops]

    out = pl.pallas_call(
        functools.partial(_net_kernel, B=B, CH=_CH),
        out_shape=jax.ShapeDtypeStruct((G, 10, B), jnp.float32),
        grid=(G,),
        in_specs=in_specs,
        out_specs=pl.BlockSpec((1, 10, B), lambda n: (n, 0, 0)),
        compiler_params=pltpu.CompilerParams(
            dimension_semantics=("parallel",)),
    )(xp, *ops)
    return jnp.transpose(out, (0, 2, 1)).reshape(N, 10)


# speedup vs baseline: 1.0802x; 1.0802x over previous
import functools

import numpy as np
import jax
import jax.numpy as jnp
from jax.experimental import pallas as pl
from jax.experimental.pallas import tpu as pltpu

_EPS = 1e-5
_B = 32
_CH = 2
_H = _W = 32


def _tap_masks(H, W, B):
    hw = H * W
    r = np.arange(hw) // W
    c = np.arange(hw) % W
    ms = []
    for dy in (-1, 0, 1):
        for dx in (-1, 0, 1):
            m = ((r + dy >= 0) & (r + dy <= H - 1)
                 & (c + dx >= 0) & (c + dx <= W - 1)).astype(np.float32)
            ms.append(np.tile(m, B))
    return np.stack(ms)


def _image_ones(hw, B):
    o = np.zeros((B * hw, B), np.float32)
    for b in range(B):
        o[b * hw:(b + 1) * hw, b] = 1.0
    return o


def _pool_sel(H, W):
    hw, wo = H * W, W // 2
    hwo = (H // 2) * wo
    sel = np.zeros((hw, hwo), np.float32)
    for q in range(hwo):
        p = (2 * (q // wo)) * W + 2 * (q % wo)
        sel[p, q] = 1.0
    return sel


_MA = _tap_masks(_H, _W, _B)
_MB = _tap_masks(_H // 2, _W // 2, _B)
_MC = _tap_masks(_H // 4, _W // 4, _B)
_OA = _image_ones(_H * _W, _B)
_OB = _image_ones((_H // 2) * (_W // 2), _B)
_OC = _image_ones((_H // 4) * (_W // 4), _B)
_S1 = _pool_sel(_H, _W)
_S2 = _pool_sel(_H // 2, _W // 2)


def _roll(x, s):
    L = x.shape[-1]
    s = s % L
    if s == 0:
        return x
    return pltpu.roll(x, shift=L - s, axis=x.ndim - 1)


def _net_kernel(x_ref, w1, w2, w3, w4, w5, w6, w7, w8, w9, w10,
                gb, pA, pB, pC, pD, pE, mA, mB, mC, oA, oB, oC, s1, s2,
                o_ref, *, B, CH):
    f32, bf16 = jnp.float32, jnp.bfloat16
    Bc = B // CH

    def expand(v, hw):
        parts = [jnp.broadcast_to(v[:, b:b + 1], (v.shape[0], hw))
                 for b in range(Bc)]
        return jnp.concatenate(parts, axis=1)

    def taps(x, msk, w_stage):
        pieces = []
        for dy in (-1, 0, 1):
            for dx in (-1, 0, 1):
                t = 3 * (dy + 1) + (dx + 1)
                v = _roll(x, dy * w_stage + dx)
                if dy != 0 or dx != 0:
                    v = v * msk[t:t + 1, :]
                pieces.append(v)
        return jnp.concatenate(pieces, axis=0)

    def conv_gn(x, wc_ref, p_ref, gn_idx, msk, onesb, hw, w_stage):
        cout = wc_ref.shape[0]
        y = jnp.dot(wc_ref[...], taps(x, msk, w_stage),
                    preferred_element_type=f32)
        yb = jnp.maximum(y.astype(bf16), 0)
        ys = jnp.concatenate([yb, yb * yb], axis=0)
        S2 = jnp.dot(ys, onesb, preferred_element_type=f32)
        stats = jnp.dot(p_ref[...],
                        jnp.concatenate([S2[:cout], S2[cout:]], axis=1),
                        preferred_element_type=f32)
        mean, e2 = stats[:, :Bc], stats[:, Bc:]
        var = jnp.maximum(e2 - mean * mean, 0.0)
        inv = jax.lax.rsqrt(var + _EPS)
        gamma = gb[0:cout, 2 * gn_idx:2 * gn_idx + 1]
        beta = gb[0:cout, 2 * gn_idx + 1:2 * gn_idx + 2]
        scale = inv * gamma
        offset = beta - mean * scale
        return yb * expand(scale.astype(bf16), hw) \
            + expand(offset.astype(bf16), hw)

    def maxpool(x, sel_ref, hw_in, w_stage):
        m1 = jnp.maximum(x, _roll(x, 1))
        m2 = jnp.maximum(m1, _roll(m1, w_stage))
        outs = [jnp.dot(m2[:, b * hw_in:(b + 1) * hw_in], sel_ref[...],
                        preferred_element_type=f32) for b in range(Bc)]
        return jnp.concatenate(outs, axis=1).astype(bf16)

    hwA, hwB, hwC = _H * _W, (_H // 2) * (_W // 2), (_H // 4) * (_W // 4)

    mAc, mBc, mCc = mA[:, :Bc * hwA], mB[:, :Bc * hwB], mC[:, :Bc * hwC]
    oAc, oBc, oCc = oA[:Bc * hwA, :Bc], oB[:Bc * hwB, :Bc], oC[:Bc * hwC, :Bc]

    def layer(fn, xs):
        return [fn(x) for x in xs]

    xin = x_ref[0]
    xs = [xin[:, c * Bc * hwA:(c + 1) * Bc * hwA] for c in range(CH)]
    xs = layer(lambda x: conv_gn(x, w1, pA, 0, mAc, oAc, hwA, _W), xs)
    xs = layer(lambda x: conv_gn(x, w2, pA, 1, mAc, oAc, hwA, _W), xs)
    xs = layer(lambda x: jnp.dot(w3[...], x,
                                 preferred_element_type=f32).astype(bf16), xs)
    xs = layer(lambda x: maxpool(x, s1, hwA, _W), xs)
    xs = layer(lambda x: conv_gn(x, w4, pB, 2, mBc, oBc, hwB, _W // 2), xs)
    xs = layer(lambda x: conv_gn(x, w5, pB, 3, mBc, oBc, hwB, _W // 2), xs)
    xs = layer(lambda x: jnp.dot(w6[...], x,
                                 preferred_element_type=f32).astype(bf16), xs)
    xs = layer(lambda x: maxpool(x, s2, hwB, _W // 2), xs)
    xs = layer(lambda x: conv_gn(x, w7, pC, 4, mCc, oCc, hwC, _W // 4), xs)
    xs = layer(lambda x: conv_gn(x, w8, pD, 5, mCc, oCc, hwC, _W // 4), xs)
    xs = layer(lambda x: conv_gn(x, w9, pE, 6, mCc, oCc, hwC, _W // 4), xs)

    def head(zf):
        g = jnp.dot(zf, oCc,
                    preferred_element_type=f32) * (1.0 / hwC)
        z = jnp.dot(w10[...], g, preferred_element_type=f32)
        m = jnp.max(z, axis=0, keepdims=True)
        lse = jnp.log(jnp.sum(jnp.exp(z - m), axis=0, keepdims=True)) + m
        return z - lse

    zs = layer(head, xs)
    o_ref[0] = zs[0] if CH == 1 else jnp.concatenate(zs, axis=1)


def _const_spec(a):
    zeros = (0,) * a.ndim
    return pl.BlockSpec(a.shape, lambda n, _z=zeros: _z)


def kernel(x, op00, op01, op02, op03, op04, op05, op06, op07, op08, op09,
           op10, op11, op12, op13, op14, op15, op16, op17, op18, op19):
    B, H, W = _B, _H, _W
    N = x.shape[0]
    G = N // B
    bf16 = jnp.bfloat16

    def wc(op):
        c9, cout, cin = op.shape
        return jnp.transpose(op, (1, 0, 2)).reshape(cout, c9 * cin).astype(bf16)

    ops = [
        wc(op00), wc(op01), op02.astype(bf16),
        wc(op03), wc(op04), op05.astype(bf16),
        wc(op06), wc(op07), wc(op08),
        op09,
        op10,
        op11, op12, op13, op14, op15,
        jnp.asarray(_MA, bf16), jnp.asarray(_MB, bf16), jnp.asarray(_MC, bf16),
        jnp.asarray(_OA, bf16), jnp.asarray(_OB, bf16), jnp.asarray(_OC, bf16),
        jnp.asarray(_S1, bf16), jnp.asarray(_S2, bf16),
    ]

    xp = jnp.pad(x.astype(bf16), ((0, 0), (0, 8 - 3), (0, 0), (0, 0)))
    xp = xp.reshape(G, B, 8, H * W)
    xp = jnp.transpose(xp, (0, 2, 1, 3)).reshape(G, 8, B * H * W)

    in_specs = [pl.BlockSpec((1, 8, B * H * W), lambda n: (n, 0, 0))]
    in_specs += [_const_spec(a) for a in ops]

    out = pl.pallas_call(
        functools.partial(_net_kernel, B=B, CH=_CH),
        out_shape=jax.ShapeDtypeStruct((G, 10, B), jnp.float32),
        grid=(G,),
        in_specs=in_specs,
        out_specs=pl.BlockSpec((1, 10, B), lambda n: (n, 0, 0)),
        compiler_params=pltpu.CompilerParams(
            dimension_semantics=("parallel",)),
    )(xp, *ops)
    return jnp.transpose(out, (0, 2, 1)).reshape(N, 10)
